# baseline (speedup 1.0000x reference)
import functools

import jax
import jax.numpy as jnp
from jax import lax
from jax.experimental import pallas as pl
from jax.experimental.pallas import tpu as pltpu
from jax.experimental.pallas import tpu_sc as plsc

_H = 64
_W = 64
_N = 4096
_ZLO = -32.0
_SCALE = 1.0 / 64.0
_BI = 1024


def _nn_mean_kernel(p_ref, mean_ref):
    i = pl.program_id(0)

    @pl.when(i == 0)
    def _():
        mean_ref[0, 0] = 0.0

    pi = p_ref[pl.ds(i * _BI, _BI), :]
    sqi = jnp.sum(pi * pi, axis=1, keepdims=True)
    row_ids = i * _BI + lax.broadcasted_iota(jnp.int32, (_BI, _BI), 0)

    def body(j, carry):
        pj = p_ref[pl.ds(j * _BI, _BI), :]
        sqj = jnp.sum(pj * pj, axis=1, keepdims=True)
        g = lax.dot_general(pi, pj, (((1,), (1,)), ((), ())),
                            preferred_element_type=jnp.float32)
        d2 = (sqi + sqj.T) - 2.0 * g
        d2 = jnp.maximum(d2, 0.0)
        col_ids = j * _BI + lax.broadcasted_iota(jnp.int32, (_BI, _BI), 1)
        d2 = jnp.where(row_ids == col_ids, jnp.inf, d2)
        return jnp.minimum(carry, jnp.min(d2, axis=1, keepdims=True))

    mind2 = lax.fori_loop(0, _N // _BI, body,
                          jnp.full((_BI, 1), jnp.inf, jnp.float32))
    nn = jnp.sqrt(mind2)
    mean_ref[0, 0] += jnp.sum(nn) * (1.0 / _N)


def _median_kernel(px_ref, py_ref, pz_ref, mean_ref, out_ref):
    r = pl.program_id(0)
    yf = r.astype(jnp.float32) - float(_H // 2)
    delta = 1.5 * (1.5 * mean_ref[0, 0])

    px = px_ref[0, :]
    py = py_ref[0, :]
    pz = pz_ref[0, :]

    base = (py >= (yf - delta)) & (py <= (yf + delta)) & (px <= (yf + delta))
    xlo = (lax.broadcasted_iota(jnp.int32, (_W, 1), 0).astype(jnp.float32)
           - float(_W // 2) - delta)
    mask = base[None, :] & (px[None, :] >= xlo)
    m = mask.astype(jnp.float32)

    cb = 128
    nchunk = _N // cb
    tri_inc = (lax.broadcasted_iota(jnp.int32, (cb, cb), 0)
               <= lax.broadcasted_iota(jnp.int32, (cb, cb), 1)
               ).astype(jnp.bfloat16)
    tri_exc = (lax.broadcasted_iota(jnp.int32, (cb, cb), 0)
               > lax.broadcasted_iota(jnp.int32, (cb, cb), 1)
               ).astype(jnp.bfloat16)
    m_bf = m.astype(jnp.bfloat16).reshape(_W * nchunk, cb)
    cumw = lax.dot_general(m_bf, tri_inc, (((1,), (0,)), ((), ())),
                           preferred_element_type=jnp.float32)
    cumw = cumw.reshape(_W, nchunk, cb)
    totals = jnp.sum(m.reshape(_W, nchunk, cb), axis=2)
    tot_pad = jnp.concatenate(
        [totals.astype(jnp.bfloat16),
         jnp.zeros((_W, cb - nchunk), jnp.bfloat16)], axis=1)
    carry = lax.dot_general(tot_pad, tri_exc.T, (((1,), (0,)), ((), ())),
                            preferred_element_type=jnp.float32)
    cum = (cumw + carry[:, :nchunk, None]).reshape(_W, _N)

    cnt = jnp.sum(totals, axis=1)
    cnt_i = cnt.astype(jnp.int32)
    lo = jnp.clip((cnt_i - 1) // 2, 0, _N - 1)
    hi = jnp.clip(cnt_i // 2, 0, _N - 1)
    rlo = (lo + 1).astype(jnp.float32)[:, None]
    rhi = (hi + 1).astype(jnp.float32)[:, None]

    ind = m * (jnp.where(cum == rlo, 0.5, 0.0) + jnp.where(cum == rhi, 0.5, 0.0))
    med = jnp.sum(ind * pz[None, :], axis=1)
    zmed = jnp.where(cnt_i > 0, (med - _ZLO) * _SCALE, 0.0)
    out_ref[0, 0, :] = zmed


def _vgather(x, idx):
    dnums = lax.GatherDimensionNumbers(
        offset_dims=(), collapsed_slice_dims=(0,), start_index_map=(0,))
    return lax.gather(x, idx[:, None], dnums, (1,),
                      mode=lax.GatherScatterMode.PROMISE_IN_BOUNDS)


_NW = 32
_ROWS_PER_W = _H // _NW
_CAP = _N + 16


def _sc_median_body(px_h, py_h, pz_h, mean_h, out_h,
                    pxv, pyv, pzv, meanv, cmv, zcv, rowv, offv_ref, lanes_ref,
                    histv):
    wid = lax.axis_index("s") * 2 + lax.axis_index("c")
    pltpu.sync_copy(px_h, pxv)
    pltpu.sync_copy(py_h, pyv)
    pltpu.sync_copy(pz_h, pzv)
    pltpu.sync_copy(mean_h, meanv)
    mean = meanv[...]
    delta = 1.5 * (1.5 * mean)
    lanes = lax.iota(jnp.int32, 16)
    lanes_ref[...] = lanes

    def do_row(rr, _):
        r = jnp.where(rr == 0, wid, _H - 1 - wid)
        yf = jnp.broadcast_to(r.astype(jnp.float32) - float(_H // 2), (16,))
        ylo = yf - delta
        yhi = yf + delta
        xhi = yf + delta
        offv_ref[...] = jnp.zeros((16,), jnp.int32)
        for q in range(_W // 16):
            histv[pl.ds(q * 16, 16)] = jnp.zeros((16,), jnp.int32)

        def pass_compress(v, carry):
            px = pxv[pl.ds(v * 16, 16)]
            py = pyv[pl.ds(v * 16, 16)]
            pz = pzv[pl.ds(v * 16, 16)]
            base = (py >= ylo) & (py <= yhi) & (px <= xhi)
            nb = plsc.all_reduce_population_count(base)

            @pl.when(jnp.max(nb) > 0)
            def _():
                cc = ((px + float(_W // 2)) + delta).astype(jnp.int32)
                up = ((cc + 1).astype(jnp.float32) - float(_W // 2)) - delta
                cc = jnp.where(up <= px, cc + 1, cc)
                dn = (cc.astype(jnp.float32) - float(_W // 2)) - delta
                cc = jnp.where(dn > px, cc - 1, cc)
                maskc = base & (cc >= 0)
                ccc = jnp.where(maskc, jnp.minimum(cc, _W - 1), -1)
                key = jnp.where(maskc, lanes, lanes + 16)
                _, perm = plsc.sort_key_val(key, lanes)
                cm_s = _vgather(ccc, perm)
                z_s = _vgather(pz, perm)
                dc, lm = plsc.scan_count(ccc, mask=maskc)
                plsc.addupdate_scatter(histv, [ccc], dc, mask=lm & maskc)
                lanesv = lanes_ref[...]
                offv = offv_ref[...]
                mtrue = lanesv >= 0
                plsc.store_scatter(cmv, [offv + lanesv], cm_s, mask=mtrue)
                plsc.store_scatter(zcv, [offv + lanesv], z_s, mask=mtrue)
                pc = plsc.all_reduce_population_count(maskc)
                offv_ref[...] = offv + pc
            return carry

        lax.fori_loop(0, _N // 16, pass_compress, 0)
        offv = offv_ref[...]
        lanesv = lanes_ref[...]
        plsc.store_scatter(cmv, [offv + lanesv],
                           jnp.full((16,), -1, jnp.int32), mask=lanesv >= 0)
        off = jnp.max(offv)
        nv = (off + 15) // 16

        cnt_vecs = []
        tot = jnp.zeros((16,), jnp.int32)
        for q in range(_W // 16 - 1, -1, -1):
            h = histv[pl.ds(q * 16, 16)]
            csum = plsc.cumsum(lax.rev(h, (0,)))
            suf = lax.rev(csum, (0,)) + tot
            tot = tot + _vgather(csum, jnp.full((16,), 15, jnp.int32))
            cnt_vecs.append(suf)
        cnt_vecs = cnt_vecs[::-1]

        for cg in range(_W // 16):
            cnt_vec = cnt_vecs[cg]
            rlo_vec = jnp.right_shift(cnt_vec - 1, 1) + 1
            rhi_vec = jnp.right_shift(cnt_vec, 1) + 1

            def do_cell(cl, out_vec, cnt_vec=cnt_vec, rlo_vec=rlo_vec,
                        rhi_vec=rhi_vec, cg=cg):
                c = jnp.broadcast_to(cg * 16 + cl, (16,))
                cls = jnp.broadcast_to(cl, (16,))
                cnt = _vgather(cnt_vec, cls)
                rlo = _vgather(rlo_vec, cls)
                rhi = _vgather(rhi_vec, cls)

                def scan_select(v, carry):
                    k, acc = carry
                    m = cmv[pl.ds(v * 16, 16)] >= c
                    zv = zcv[pl.ds(v * 16, 16)]
                    cum = plsc.cumsum(jnp.where(m, 1, 0)) + k
                    w = (jnp.where(m & (cum == rlo), 0.5, 0.0)
                         + jnp.where(m & (cum == rhi), 0.5, 0.0))
                    return (k + plsc.all_reduce_population_count(m),
                            acc + zv * w)

                _, acc = lax.fori_loop(
                    0, nv, scan_select,
                    (jnp.zeros((16,), jnp.int32), jnp.zeros((16,), jnp.float32)))
                med = jnp.broadcast_to(jnp.sum(acc), (16,))
                val = jnp.where(cnt > 0, (med - _ZLO) * _SCALE, 0.0)
                return jnp.where(lanes == cl, val, out_vec)

            out_vec = lax.fori_loop(0, 16, do_cell,
                                    jnp.zeros((16,), jnp.float32))
            rowv[pl.ds(cg * 16, 16)] = out_vec
        pltpu.sync_copy(rowv, out_h.at[r])
        return 0

    lax.fori_loop(0, _ROWS_PER_W, do_row, 0)


def _sc_median(px, py, pz, mean16):
    fn = pl.kernel(
        _sc_median_body,
        out_type=jax.ShapeDtypeStruct((_H, _W), jnp.float32),
        mesh=plsc.VectorSubcoreMesh(core_axis_name="c", subcore_axis_name="s",
                                    num_cores=2, num_subcores=16),
        scratch_types=[
            pltpu.VMEM((_N,), jnp.float32),
            pltpu.VMEM((_N,), jnp.float32),
            pltpu.VMEM((_N,), jnp.float32),
            pltpu.VMEM((16,), jnp.float32),
            pltpu.VMEM((_CAP,), jnp.int32),
            pltpu.VMEM((_CAP,), jnp.float32),
            pltpu.VMEM((_W,), jnp.float32),
            pltpu.VMEM((16,), jnp.int32),
            pltpu.VMEM((16,), jnp.int32),
            pltpu.VMEM((_W,), jnp.int32),
        ],
        compiler_params=pltpu.CompilerParams(needs_layout_passes=False),
    )
    return fn(px, py, pz, mean16)


def kernel(points):
    pcd = points * float(_H)
    p_pad = jnp.zeros((_N, 128), jnp.float32).at[:, :3].set(pcd)

    mean_nn = pl.pallas_call(
        _nn_mean_kernel,
        grid=(_N // _BI,),
        in_specs=[pl.BlockSpec((_N, 128), lambda i: (0, 0))],
        out_specs=pl.BlockSpec(memory_space=pltpu.SMEM),
        out_shape=jax.ShapeDtypeStruct((1, 1), jnp.float32),
    )(p_pad)

    pz_s, px_s, py_s = lax.sort(
        (pcd[:, 2], pcd[:, 0], pcd[:, 1]), num_keys=1)
    mean16 = mean_nn.reshape(1) * jnp.ones((16,), jnp.float32)
    return _sc_median(px_s, py_s, pz_s, mean16)


# speedup vs baseline: 1.0903x; 1.0903x over previous
import functools

import jax
import jax.numpy as jnp
from jax import lax
from jax.experimental import pallas as pl
from jax.experimental.pallas import tpu as pltpu
from jax.experimental.pallas import tpu_sc as plsc

_H = 64
_W = 64
_N = 4096
_ZLO = -32.0
_SCALE = 1.0 / 64.0
_BI = 1024


def _nn_mean_kernel(p_ref, mean_ref):
    i = pl.program_id(0)

    @pl.when(i == 0)
    def _():
        mean_ref[0, 0] = 0.0

    pi = p_ref[pl.ds(i * _BI, _BI), :]
    sqi = jnp.sum(pi * pi, axis=1, keepdims=True)
    row_ids = i * _BI + lax.broadcasted_iota(jnp.int32, (_BI, _BI), 0)

    def body(j, carry):
        pj = p_ref[pl.ds(j * _BI, _BI), :]
        sqj = jnp.sum(pj * pj, axis=1, keepdims=True)
        g = lax.dot_general(pi, pj, (((1,), (1,)), ((), ())),
                            preferred_element_type=jnp.float32)
        d2 = (sqi + sqj.T) - 2.0 * g
        d2 = jnp.maximum(d2, 0.0)
        col_ids = j * _BI + lax.broadcasted_iota(jnp.int32, (_BI, _BI), 1)
        d2 = jnp.where(row_ids == col_ids, jnp.inf, d2)
        return jnp.minimum(carry, jnp.min(d2, axis=1, keepdims=True))

    mind2 = lax.fori_loop(0, _N // _BI, body,
                          jnp.full((_BI, 1), jnp.inf, jnp.float32))
    nn = jnp.sqrt(mind2)
    mean_ref[0, 0] += jnp.sum(nn) * (1.0 / _N)


def _median_kernel(px_ref, py_ref, pz_ref, mean_ref, out_ref):
    r = pl.program_id(0)
    yf = r.astype(jnp.float32) - float(_H // 2)
    delta = 1.5 * (1.5 * mean_ref[0, 0])

    px = px_ref[0, :]
    py = py_ref[0, :]
    pz = pz_ref[0, :]

    base = (py >= (yf - delta)) & (py <= (yf + delta)) & (px <= (yf + delta))
    xlo = (lax.broadcasted_iota(jnp.int32, (_W, 1), 0).astype(jnp.float32)
           - float(_W // 2) - delta)
    mask = base[None, :] & (px[None, :] >= xlo)
    m = mask.astype(jnp.float32)

    cb = 128
    nchunk = _N // cb
    tri_inc = (lax.broadcasted_iota(jnp.int32, (cb, cb), 0)
               <= lax.broadcasted_iota(jnp.int32, (cb, cb), 1)
               ).astype(jnp.bfloat16)
    tri_exc = (lax.broadcasted_iota(jnp.int32, (cb, cb), 0)
               > lax.broadcasted_iota(jnp.int32, (cb, cb), 1)
               ).astype(jnp.bfloat16)
    m_bf = m.astype(jnp.bfloat16).reshape(_W * nchunk, cb)
    cumw = lax.dot_general(m_bf, tri_inc, (((1,), (0,)), ((), ())),
                           preferred_element_type=jnp.float32)
    cumw = cumw.reshape(_W, nchunk, cb)
    totals = jnp.sum(m.reshape(_W, nchunk, cb), axis=2)
    tot_pad = jnp.concatenate(
        [totals.astype(jnp.bfloat16),
         jnp.zeros((_W, cb - nchunk), jnp.bfloat16)], axis=1)
    carry = lax.dot_general(tot_pad, tri_exc.T, (((1,), (0,)), ((), ())),
                            preferred_element_type=jnp.float32)
    cum = (cumw + carry[:, :nchunk, None]).reshape(_W, _N)

    cnt = jnp.sum(totals, axis=1)
    cnt_i = cnt.astype(jnp.int32)
    lo = jnp.clip((cnt_i - 1) // 2, 0, _N - 1)
    hi = jnp.clip(cnt_i // 2, 0, _N - 1)
    rlo = (lo + 1).astype(jnp.float32)[:, None]
    rhi = (hi + 1).astype(jnp.float32)[:, None]

    ind = m * (jnp.where(cum == rlo, 0.5, 0.0) + jnp.where(cum == rhi, 0.5, 0.0))
    med = jnp.sum(ind * pz[None, :], axis=1)
    zmed = jnp.where(cnt_i > 0, (med - _ZLO) * _SCALE, 0.0)
    out_ref[0, 0, :] = zmed


def _vgather(x, idx):
    dnums = lax.GatherDimensionNumbers(
        offset_dims=(), collapsed_slice_dims=(0,), start_index_map=(0,))
    return lax.gather(x, idx[:, None], dnums, (1,),
                      mode=lax.GatherScatterMode.PROMISE_IN_BOUNDS)


_NW = 32
_ROWS_PER_W = _H // _NW
_CAP = _N + 16


def _sc_median_body(px_h, py_h, pz_h, mean_h, out_h,
                    pxv, pyv, pzv, meanv, cmv, zcv, rowv, offv_ref, lanes_ref,
                    histv):
    wid = lax.axis_index("s") * 2 + lax.axis_index("c")
    pltpu.sync_copy(px_h, pxv)
    pltpu.sync_copy(py_h, pyv)
    pltpu.sync_copy(pz_h, pzv)
    pltpu.sync_copy(mean_h, meanv)
    mean = meanv[...]
    delta = 1.5 * (1.5 * mean)
    lanes = lax.iota(jnp.int32, 16)
    lanes_ref[...] = lanes

    def do_row(rr, _):
        r = jnp.where(rr == 0, wid, _H - 1 - wid)
        yf = jnp.broadcast_to(r.astype(jnp.float32) - float(_H // 2), (16,))
        ylo = yf - delta
        yhi = yf + delta
        xhi = yf + delta
        offv_ref[...] = jnp.zeros((16,), jnp.int32)
        for q in range(_W // 16):
            histv[pl.ds(q * 16, 16)] = jnp.zeros((16,), jnp.int32)

        def pass_compress(v, carry):
            px = pxv[pl.ds(v * 16, 16)]
            py = pyv[pl.ds(v * 16, 16)]
            pz = pzv[pl.ds(v * 16, 16)]
            base = (py >= ylo) & (py <= yhi) & (px <= xhi)
            cc = ((px + float(_W // 2)) + delta).astype(jnp.int32)
            up = ((cc + 1).astype(jnp.float32) - float(_W // 2)) - delta
            cc = jnp.where(up <= px, cc + 1, cc)
            dn = (cc.astype(jnp.float32) - float(_W // 2)) - delta
            cc = jnp.where(dn > px, cc - 1, cc)
            maskc = base & (cc >= 0)
            ccc = jnp.where(maskc, jnp.minimum(cc, _W - 1), -1)
            key = jnp.where(maskc, lanes, lanes + 16)
            _, perm = plsc.sort_key_val(key, lanes)
            cm_s = _vgather(ccc, perm)
            z_s = _vgather(pz, perm)
            dc, lm = plsc.scan_count(ccc, mask=maskc)
            plsc.addupdate_scatter(histv, [ccc], dc, mask=lm & maskc)
            lanesv = lanes_ref[...]
            offv = offv_ref[...]
            mtrue = lanesv >= 0
            plsc.store_scatter(cmv, [offv + lanesv], cm_s, mask=mtrue)
            plsc.store_scatter(zcv, [offv + lanesv], z_s, mask=mtrue)
            pc = plsc.all_reduce_population_count(maskc)
            offv_ref[...] = offv + pc
            return carry

        lax.fori_loop(0, _N // 16, pass_compress, 0)
        offv = offv_ref[...]
        lanesv = lanes_ref[...]
        plsc.store_scatter(cmv, [offv + lanesv],
                           jnp.full((16,), -1, jnp.int32), mask=lanesv >= 0)
        off = jnp.max(offv)
        nv = (off + 15) // 16

        cnt_vecs = []
        tot = jnp.zeros((16,), jnp.int32)
        for q in range(_W // 16 - 1, -1, -1):
            h = histv[pl.ds(q * 16, 16)]
            csum = plsc.cumsum(lax.rev(h, (0,)))
            suf = lax.rev(csum, (0,)) + tot
            tot = tot + _vgather(csum, jnp.full((16,), 15, jnp.int32))
            cnt_vecs.append(suf)
        cnt_vecs = cnt_vecs[::-1]

        for cg in range(_W // 16):
            cnt_vec = cnt_vecs[cg]
            rlo_vec = jnp.right_shift(cnt_vec - 1, 1) + 1
            rhi_vec = jnp.right_shift(cnt_vec, 1) + 1

            def do_cell(cl, out_vec, cnt_vec=cnt_vec, rlo_vec=rlo_vec,
                        rhi_vec=rhi_vec, cg=cg):
                c = jnp.broadcast_to(cg * 16 + cl, (16,))
                cls = jnp.broadcast_to(cl, (16,))
                cnt = _vgather(cnt_vec, cls)
                rlo = _vgather(rlo_vec, cls)
                rhi = _vgather(rhi_vec, cls)

                def scan_select(v, carry):
                    k, acc = carry
                    m = cmv[pl.ds(v * 16, 16)] >= c
                    zv = zcv[pl.ds(v * 16, 16)]
                    cum = plsc.cumsum(jnp.where(m, 1, 0)) + k
                    w = (jnp.where(m & (cum == rlo), 0.5, 0.0)
                         + jnp.where(m & (cum == rhi), 0.5, 0.0))
                    return (k + plsc.all_reduce_population_count(m),
                            acc + zv * w)

                _, acc = lax.fori_loop(
                    0, nv, scan_select,
                    (jnp.zeros((16,), jnp.int32), jnp.zeros((16,), jnp.float32)))
                med = jnp.broadcast_to(jnp.sum(acc), (16,))
                val = jnp.where(cnt > 0, (med - _ZLO) * _SCALE, 0.0)
                return jnp.where(lanes == cl, val, out_vec)

            out_vec = lax.fori_loop(0, 16, do_cell,
                                    jnp.zeros((16,), jnp.float32))
            rowv[pl.ds(cg * 16, 16)] = out_vec
        pltpu.sync_copy(rowv, out_h.at[r])
        return 0

    lax.fori_loop(0, _ROWS_PER_W, do_row, 0)


def _sc_median(px, py, pz, mean16):
    fn = pl.kernel(
        _sc_median_body,
        out_type=jax.ShapeDtypeStruct((_H, _W), jnp.float32),
        mesh=plsc.VectorSubcoreMesh(core_axis_name="c", subcore_axis_name="s",
                                    num_cores=2, num_subcores=16),
        scratch_types=[
            pltpu.VMEM((_N,), jnp.float32),
            pltpu.VMEM((_N,), jnp.float32),
            pltpu.VMEM((_N,), jnp.float32),
            pltpu.VMEM((16,), jnp.float32),
            pltpu.VMEM((_CAP,), jnp.int32),
            pltpu.VMEM((_CAP,), jnp.float32),
            pltpu.VMEM((_W,), jnp.float32),
            pltpu.VMEM((16,), jnp.int32),
            pltpu.VMEM((16,), jnp.int32),
            pltpu.VMEM((_W,), jnp.int32),
        ],
        compiler_params=pltpu.CompilerParams(needs_layout_passes=False),
    )
    return fn(px, py, pz, mean16)


def kernel(points):
    pcd = points * float(_H)
    p_pad = jnp.zeros((_N, 128), jnp.float32).at[:, :3].set(pcd)

    mean_nn = pl.pallas_call(
        _nn_mean_kernel,
        grid=(_N // _BI,),
        in_specs=[pl.BlockSpec((_N, 128), lambda i: (0, 0))],
        out_specs=pl.BlockSpec(memory_space=pltpu.SMEM),
        out_shape=jax.ShapeDtypeStruct((1, 1), jnp.float32),
    )(p_pad)

    pz_s, px_s, py_s = lax.sort(
        (pcd[:, 2], pcd[:, 0], pcd[:, 1]), num_keys=1)
    mean16 = mean_nn.reshape(1) * jnp.ones((16,), jnp.float32)
    return _sc_median(px_s, py_s, pz_s, mean16)
